# baseline (speedup 1.0000x reference)
import jax
import jax.numpy as jnp
from jax.experimental import pallas as pl
from jax.experimental.pallas import tpu as pltpu

_K = 16
_ROWS = 128


def _ce(v1, v2, i1, i2):
    c = v1 <= v2
    return (jnp.minimum(v1, v2), jnp.maximum(v1, v2),
            jnp.where(c, i1, i2), jnp.where(c, i2, i1))


def _sort4(v, i):
    (v0, v1, v2, v3) = v
    (i0, i1, i2, i3) = i
    v0, v1, i0, i1 = _ce(v0, v1, i0, i1)
    v2, v3, i2, i3 = _ce(v2, v3, i2, i3)
    v0, v2, i0, i2 = _ce(v0, v2, i0, i2)
    v1, v3, i1, i3 = _ce(v1, v3, i1, i3)
    v1, v2, i1, i2 = _ce(v1, v2, i1, i2)
    return (v0, v1, v2, v3), (i0, i1, i2, i3)


def _half_ce(va, vb, ia, ib):
    c = va <= vb
    return jnp.minimum(va, vb), jnp.where(c, ia, ib)


def _bitonic4_cleanup(lv, li):
    l0, l1, l2, l3 = lv
    i0, i1, i2, i3 = li
    l0, l2, i0, i2 = _ce(l0, l2, i0, i2)
    l1, l3, i1, i3 = _ce(l1, l3, i1, i3)
    l0, l1, i0, i1 = _ce(l0, l1, i0, i1)
    l2, l3, i2, i3 = _ce(l2, l3, i2, i3)
    return [l0, l1, l2, l3], [i0, i1, i2, i3]


def _merge_low4(av, ai, bv, bi):
    lv, li = [], []
    for r in range(4):
        v, i = _half_ce(av[r], bv[3 - r], ai[r], bi[3 - r])
        lv.append(v)
        li.append(i)
    return _bitonic4_cleanup(lv, li)


def _knn_body(xr_ref, xf_ref, dists_ref, idx_ref):
    xr = xr_ref[...]
    xf = xf_ref[...]
    inner = -2.0 * jax.lax.dot_general(
        xr.astype(jnp.bfloat16), xf.astype(jnp.bfloat16),
        (((1,), (1,)), ((), ())),
        preferred_element_type=jnp.float32,
    )
    xx_r = jnp.sum(xr * xr, axis=1, keepdims=True)
    xx_c = jnp.sum(xf * xf, axis=1)
    pd = -xx_r - inner - xx_c[None, :]

    fiota = jax.lax.broadcasted_iota(jnp.int32, pd.shape, 1).astype(jnp.float32)
    inf = jnp.float32(jnp.inf)

    w = pd.shape[1] // 8
    xs = [pd[:, t * w:(t + 1) * w] for t in range(8)]
    js = [fiota[:, t * w:(t + 1) * w] for t in range(8)]
    av, ai = _sort4(xs[:4], js[:4])
    bv, bi = _sort4(xs[4:], js[4:])
    sv, si = _merge_low4(list(av), list(ai), list(bv), list(bi))
    for _ in range(2):
        h = sv[0].shape[1] // 2
        sv, si = _merge_low4([s[:, :h] for s in sv], [s[:, :h] for s in si],
                             [s[:, h:] for s in sv], [s[:, h:] for s in si])

    s0, s1, s2, s3 = sv
    i0, i1, i2, i3 = si
    vals = []
    inds = []
    for k in range(_K + 1):
        m = jnp.min(s0, axis=1, keepdims=True)
        ind = jnp.min(jnp.where(s0 == m, i0, inf), axis=1, keepdims=True)
        vals.append(m)
        inds.append(ind)
        if k < _K:
            hit = i0 == ind
            s0 = jnp.where(hit, s1, s0)
            i0 = jnp.where(hit, i1, i0)
            s1 = jnp.where(hit, s2, s1)
            i1 = jnp.where(hit, i2, i1)
            s2 = jnp.where(hit, s3, s2)
            i2 = jnp.where(hit, i3, i2)
            s3 = jnp.where(hit, inf, s3)
    dists_ref[...] = jnp.concatenate(vals, axis=1)
    idx_ref[...] = jnp.concatenate(inds, axis=1).astype(jnp.int32)


def kernel(x):
    b, npts, d = x.shape
    n = b * npts
    xf = x.reshape(n, d)
    grid = n // _ROWS
    dists, idx = pl.pallas_call(
        _knn_body,
        grid=(grid,),
        in_specs=[
            pl.BlockSpec((_ROWS, d), lambda i: (i, 0)),
            pl.BlockSpec((n, d), lambda i: (0, 0)),
        ],
        out_specs=[
            pl.BlockSpec((_ROWS, _K + 1), lambda i: (i, 0)),
            pl.BlockSpec((_ROWS, _K + 1), lambda i: (i, 0)),
        ],
        out_shape=[
            jax.ShapeDtypeStruct((n, _K + 1), jnp.float32),
            jax.ShapeDtypeStruct((n, _K + 1), jnp.int32),
        ],
        compiler_params=pltpu.CompilerParams(
            dimension_semantics=("arbitrary",),
        ),
    )(xf, xf)
    return (
        dists[:, 1:].reshape(b, npts, _K),
        idx[:, 1:].reshape(b, npts, _K),
    )


# speedup vs baseline: 1.1145x; 1.1145x over previous
import jax
import jax.numpy as jnp
from jax.experimental import pallas as pl
from jax.experimental.pallas import tpu as pltpu

_K = 16
_ROWS = 256


def _ce(v1, v2, i1, i2):
    c = v1 <= v2
    return (jnp.minimum(v1, v2), jnp.maximum(v1, v2),
            jnp.where(c, i1, i2), jnp.where(c, i2, i1))


def _sort4(v, i):
    (v0, v1, v2, v3) = v
    (i0, i1, i2, i3) = i
    v0, v1, i0, i1 = _ce(v0, v1, i0, i1)
    v2, v3, i2, i3 = _ce(v2, v3, i2, i3)
    v0, v2, i0, i2 = _ce(v0, v2, i0, i2)
    v1, v3, i1, i3 = _ce(v1, v3, i1, i3)
    v1, v2, i1, i2 = _ce(v1, v2, i1, i2)
    return (v0, v1, v2, v3), (i0, i1, i2, i3)


def _half_ce(va, vb, ia, ib):
    c = va <= vb
    return jnp.minimum(va, vb), jnp.where(c, ia, ib)


def _bitonic4_cleanup(lv, li):
    l0, l1, l2, l3 = lv
    i0, i1, i2, i3 = li
    l0, l2, i0, i2 = _ce(l0, l2, i0, i2)
    l1, l3, i1, i3 = _ce(l1, l3, i1, i3)
    l0, l1, i0, i1 = _ce(l0, l1, i0, i1)
    l2, l3, i2, i3 = _ce(l2, l3, i2, i3)
    return [l0, l1, l2, l3], [i0, i1, i2, i3]


def _merge_low4(av, ai, bv, bi):
    lv, li = [], []
    for r in range(4):
        v, i = _half_ce(av[r], bv[3 - r], ai[r], bi[3 - r])
        lv.append(v)
        li.append(i)
    return _bitonic4_cleanup(lv, li)


def _knn_body(xr_ref, xf_ref, dists_ref, idx_ref):
    xr = xr_ref[...]
    xf = xf_ref[...]
    inner = -2.0 * jax.lax.dot_general(
        xr.astype(jnp.bfloat16), xf.astype(jnp.bfloat16),
        (((1,), (1,)), ((), ())),
        preferred_element_type=jnp.float32,
    )
    xx_r = jnp.sum(xr * xr, axis=1, keepdims=True)
    xx_c = jnp.sum(xf * xf, axis=1)
    pd = -xx_r - inner - xx_c[None, :]

    fiota = jax.lax.broadcasted_iota(jnp.int32, pd.shape, 1).astype(jnp.float32)
    inf = jnp.float32(jnp.inf)

    w = pd.shape[1] // 8
    xs = [pd[:, t * w:(t + 1) * w] for t in range(8)]
    js = [fiota[:, t * w:(t + 1) * w] for t in range(8)]
    av, ai = _sort4(xs[:4], js[:4])
    bv, bi = _sort4(xs[4:], js[4:])
    sv, si = _merge_low4(list(av), list(ai), list(bv), list(bi))
    for _ in range(2):
        h = sv[0].shape[1] // 2
        sv, si = _merge_low4([s[:, :h] for s in sv], [s[:, :h] for s in si],
                             [s[:, h:] for s in sv], [s[:, h:] for s in si])

    s0, s1, s2, s3 = sv
    i0, i1, i2, i3 = si
    vals = []
    inds = []
    for k in range(_K + 1):
        m = jnp.min(s0, axis=1, keepdims=True)
        ind = jnp.min(jnp.where(s0 == m, i0, inf), axis=1, keepdims=True)
        vals.append(m)
        inds.append(ind)
        if k < _K:
            hit = i0 == ind
            s0 = jnp.where(hit, s1, s0)
            i0 = jnp.where(hit, i1, i0)
            s1 = jnp.where(hit, s2, s1)
            i1 = jnp.where(hit, i2, i1)
            s2 = jnp.where(hit, s3, s2)
            i2 = jnp.where(hit, i3, i2)
            s3 = jnp.where(hit, inf, s3)
    dists_ref[...] = jnp.concatenate(vals, axis=1)
    idx_ref[...] = jnp.concatenate(inds, axis=1).astype(jnp.int32)


def kernel(x):
    b, npts, d = x.shape
    n = b * npts
    xf = x.reshape(n, d)
    grid = n // _ROWS
    dists, idx = pl.pallas_call(
        _knn_body,
        grid=(grid,),
        in_specs=[
            pl.BlockSpec((_ROWS, d), lambda i: (i, 0)),
            pl.BlockSpec((n, d), lambda i: (0, 0)),
        ],
        out_specs=[
            pl.BlockSpec((_ROWS, _K + 1), lambda i: (i, 0)),
            pl.BlockSpec((_ROWS, _K + 1), lambda i: (i, 0)),
        ],
        out_shape=[
            jax.ShapeDtypeStruct((n, _K + 1), jnp.float32),
            jax.ShapeDtypeStruct((n, _K + 1), jnp.int32),
        ],
        compiler_params=pltpu.CompilerParams(
            dimension_semantics=("arbitrary",),
        ),
    )(xf, xf)
    return (
        dists[:, 1:].reshape(b, npts, _K),
        idx[:, 1:].reshape(b, npts, _K),
    )


# speedup vs baseline: 1.3780x; 1.2364x over previous
import jax
import jax.numpy as jnp
from jax.experimental import pallas as pl
from jax.experimental.pallas import tpu as pltpu

_K = 16
_ROWS = 512


def _ce(v1, v2, i1, i2):
    c = v1 <= v2
    return (jnp.minimum(v1, v2), jnp.maximum(v1, v2),
            jnp.where(c, i1, i2), jnp.where(c, i2, i1))


def _sort4(v, i):
    (v0, v1, v2, v3) = v
    (i0, i1, i2, i3) = i
    v0, v1, i0, i1 = _ce(v0, v1, i0, i1)
    v2, v3, i2, i3 = _ce(v2, v3, i2, i3)
    v0, v2, i0, i2 = _ce(v0, v2, i0, i2)
    v1, v3, i1, i3 = _ce(v1, v3, i1, i3)
    v1, v2, i1, i2 = _ce(v1, v2, i1, i2)
    return (v0, v1, v2, v3), (i0, i1, i2, i3)


def _half_ce(va, vb, ia, ib):
    c = va <= vb
    return jnp.minimum(va, vb), jnp.where(c, ia, ib)


def _bitonic4_cleanup(lv, li):
    l0, l1, l2, l3 = lv
    i0, i1, i2, i3 = li
    l0, l2, i0, i2 = _ce(l0, l2, i0, i2)
    l1, l3, i1, i3 = _ce(l1, l3, i1, i3)
    l0, l1, i0, i1 = _ce(l0, l1, i0, i1)
    l2, l3, i2, i3 = _ce(l2, l3, i2, i3)
    return [l0, l1, l2, l3], [i0, i1, i2, i3]


def _merge_low4(av, ai, bv, bi):
    lv, li = [], []
    for r in range(4):
        v, i = _half_ce(av[r], bv[3 - r], ai[r], bi[3 - r])
        lv.append(v)
        li.append(i)
    return _bitonic4_cleanup(lv, li)


def _knn_body(xr_ref, xf_ref, dists_ref, idx_ref):
    xr = xr_ref[...]
    xf = xf_ref[...]
    inner = -2.0 * jax.lax.dot_general(
        xr.astype(jnp.bfloat16), xf.astype(jnp.bfloat16),
        (((1,), (1,)), ((), ())),
        preferred_element_type=jnp.float32,
    )
    xx_r = jnp.sum(xr * xr, axis=1, keepdims=True)
    xx_c = jnp.sum(xf * xf, axis=1)
    pd = -xx_r - inner - xx_c[None, :]

    fiota = jax.lax.broadcasted_iota(jnp.int32, pd.shape, 1).astype(jnp.float32)
    inf = jnp.float32(jnp.inf)

    w = pd.shape[1] // 8
    xs = [pd[:, t * w:(t + 1) * w] for t in range(8)]
    js = [fiota[:, t * w:(t + 1) * w] for t in range(8)]
    av, ai = _sort4(xs[:4], js[:4])
    bv, bi = _sort4(xs[4:], js[4:])
    sv, si = _merge_low4(list(av), list(ai), list(bv), list(bi))
    for _ in range(2):
        h = sv[0].shape[1] // 2
        sv, si = _merge_low4([s[:, :h] for s in sv], [s[:, :h] for s in si],
                             [s[:, h:] for s in sv], [s[:, h:] for s in si])

    s0, s1, s2, s3 = sv
    i0, i1, i2, i3 = si
    vals = []
    inds = []
    for k in range(_K + 1):
        m = jnp.min(s0, axis=1, keepdims=True)
        ind = jnp.min(jnp.where(s0 == m, i0, inf), axis=1, keepdims=True)
        vals.append(m)
        inds.append(ind)
        if k < _K:
            hit = i0 == ind
            s0 = jnp.where(hit, s1, s0)
            i0 = jnp.where(hit, i1, i0)
            s1 = jnp.where(hit, s2, s1)
            i1 = jnp.where(hit, i2, i1)
            s2 = jnp.where(hit, s3, s2)
            i2 = jnp.where(hit, i3, i2)
            s3 = jnp.where(hit, inf, s3)
    dists_ref[...] = jnp.concatenate(vals, axis=1)
    idx_ref[...] = jnp.concatenate(inds, axis=1).astype(jnp.int32)


def kernel(x):
    b, npts, d = x.shape
    n = b * npts
    xf = x.reshape(n, d)
    grid = n // _ROWS
    dists, idx = pl.pallas_call(
        _knn_body,
        grid=(grid,),
        in_specs=[
            pl.BlockSpec((_ROWS, d), lambda i: (i, 0)),
            pl.BlockSpec((n, d), lambda i: (0, 0)),
        ],
        out_specs=[
            pl.BlockSpec((_ROWS, _K + 1), lambda i: (i, 0)),
            pl.BlockSpec((_ROWS, _K + 1), lambda i: (i, 0)),
        ],
        out_shape=[
            jax.ShapeDtypeStruct((n, _K + 1), jnp.float32),
            jax.ShapeDtypeStruct((n, _K + 1), jnp.int32),
        ],
        compiler_params=pltpu.CompilerParams(
            dimension_semantics=("arbitrary",),
        ),
    )(xf, xf)
    return (
        dists[:, 1:].reshape(b, npts, _K),
        idx[:, 1:].reshape(b, npts, _K),
    )
